# baseline (speedup 1.0000x reference)
import functools
import jax
import jax.numpy as jnp
from jax import lax
from jax.experimental import pallas as pl
from jax.experimental.pallas import tpu as pltpu
from jax.experimental.pallas import tpu_sc as plsc

N, S, D = 16384, 50, 64
MAX_POS = 64
NC, NS = 2, 16
NW = NC * NS
ROWS_PER_W = N // NW
REP = 16
CHUNKS = ROWS_PER_W // REP
DEPTH = 8
LANES = 16

_mesh = plsc.VectorSubcoreMesh(core_axis_name="c", subcore_axis_name="s")


@functools.partial(
    pl.kernel,
    mesh=_mesh,
    out_type=jax.ShapeDtypeStruct((N, S, D), jnp.float32),
    scratch_types=[
        pltpu.VMEM((REP, S, D), jnp.float32),
        pltpu.VMEM((MAX_POS, D), jnp.float32),
        pltpu.SemaphoreType.DMA,
    ],
)
def _sc_broadcast(table_hbm, out_hbm, buf, buf64, sem):
    wid = lax.axis_index("s") * NC + lax.axis_index("c")
    base = wid * ROWS_PER_W
    pltpu.sync_copy(table_hbm, buf64)
    for r in range(S):
        for j in range(D // LANES):
            buf[0, r, pl.ds(j * LANES, LANES)] = buf64[r, pl.ds(j * LANES, LANES)]

    def _replicate(rep, carry):
        for r in range(S):
            for j in range(D // LANES):
                buf[rep, r, pl.ds(j * LANES, LANES)] = buf[
                    0, r, pl.ds(j * LANES, LANES)
                ]
        return carry

    lax.fori_loop(1, REP, _replicate, 0)
    handles = []
    for c in range(CHUNKS):
        handles.append(
            pltpu.async_copy(buf, out_hbm.at[pl.ds(base + c * REP, REP)], sem)
        )
        if len(handles) > DEPTH:
            handles[len(handles) - 1 - DEPTH].wait()
    for h in handles[-DEPTH:]:
        h.wait()


def kernel(x, table):
    del x
    return _sc_broadcast(table)


# speedup vs baseline: 1.0133x; 1.0133x over previous
import functools
import jax
import jax.numpy as jnp
from jax import lax
from jax.experimental import pallas as pl
from jax.experimental.pallas import tpu as pltpu
from jax.experimental.pallas import tpu_sc as plsc

N, S, D = 16384, 50, 64
MAX_POS = 64
NC, NS = 2, 16
NW = NC * NS
ROWS_PER_W = N // NW
REP = 8
CHUNKS = ROWS_PER_W // REP
DEPTH = 8
LANES = 16

_mesh = plsc.VectorSubcoreMesh(core_axis_name="c", subcore_axis_name="s")


@functools.partial(
    pl.kernel,
    mesh=_mesh,
    out_type=jax.ShapeDtypeStruct((N, S, D), jnp.float32),
    scratch_types=[
        pltpu.VMEM((REP, S, D), jnp.float32),
        pltpu.VMEM((MAX_POS, D), jnp.float32),
        pltpu.SemaphoreType.DMA,
    ],
    compiler_params=pltpu.CompilerParams(use_tc_tiling_on_sc=True),
)
def _sc_broadcast(table_hbm, out_hbm, buf, buf64, sem):
    wid = lax.axis_index("s") * NC + lax.axis_index("c")
    base = wid * ROWS_PER_W
    pltpu.sync_copy(table_hbm, buf64)
    for r in range(S):
        for j in range(D // LANES):
            buf[0, r, pl.ds(j * LANES, LANES)] = buf64[r, pl.ds(j * LANES, LANES)]

    def _replicate(rep, carry):
        for r in range(S):
            for j in range(D // LANES):
                buf[rep, r, pl.ds(j * LANES, LANES)] = buf[
                    0, r, pl.ds(j * LANES, LANES)
                ]
        return carry

    lax.fori_loop(1, REP, _replicate, 0)
    handles = []
    for c in range(CHUNKS):
        handles.append(
            pltpu.async_copy(buf, out_hbm.at[pl.ds(base + c * REP, REP)], sem)
        )
        if len(handles) > DEPTH:
            handles[len(handles) - 1 - DEPTH].wait()
    for h in handles[-DEPTH:]:
        h.wait()


def kernel(x, table):
    del x
    return _sc_broadcast(table)
